# baseline (speedup 1.0000x reference)
import functools

import jax
import jax.numpy as jnp
from jax import lax
from jax.experimental import pallas as pl
from jax.experimental.pallas import tpu as pltpu
from jax.experimental.pallas import tpu_sc as plsc

C = 256
D = H = W = 64
V = D * H * W
BN = 64 * 512
L = 16
NC, NS = 2, 16
NW = NC * NS
BPW = BN // NW
CHUNK = 128
NCHUNK = BPW // CHUNK
NBUF = 2
NROUND = NCHUNK // NBUF
LOOKAHEAD = 1


def _body(pts_hbm, lab_hbm, pemb_hbm, table_hbm, out_hbm,
          zv, yv, xv, idx_v, lab_v, pemb_v, rows_v, semc, semg, semo):
    wid = lax.axis_index("s") * NC + lax.axis_index("c")
    base = wid * BPW

    cz = pltpu.async_copy(pts_hbm.at[pl.ds(base, BPW)], zv, semc[0])
    cy = pltpu.async_copy(pts_hbm.at[pl.ds(BN + base, BPW)], yv, semc[1])
    cx = pltpu.async_copy(pts_hbm.at[pl.ds(2 * BN + base, BPW)], xv, semc[2])
    cl = pltpu.async_copy(lab_hbm.at[pl.ds(base, BPW)], lab_v, semc[3])
    cp = pltpu.async_copy(pemb_hbm, pemb_v, semc[4])
    cz.wait(); cy.wait(); cx.wait()

    def flat_body(i, _):
        s = pl.ds(i * L, L)
        z = jnp.clip(zv[s], 0, D - 1)
        y = jnp.clip(yv[s], 0, H - 1)
        x = jnp.clip(xv[s], 0, W - 1)
        idx_v[s] = (z * H + y) * W + x
        return _

    lax.fori_loop(0, BPW // L, flat_body, None)
    cl.wait(); cp.wait()

    e0 = [pemb_v[0, pl.ds(j * L, L)] for j in range(C // L)]
    e1 = [pemb_v[1, pl.ds(j * L, L)] for j in range(C // L)]

    def start_gather(c, b):
        off = pl.multiple_of(c * CHUNK, CHUNK)
        idx_slice = idx_v.at[pl.ds(off, CHUNK)]
        return pltpu.async_copy(table_hbm.at[idx_slice], rows_v.at[b], semg[b])

    def start_scatter(c, b):
        off = pl.multiple_of(base + c * CHUNK, CHUNK)
        return pltpu.async_copy(rows_v.at[b], out_hbm.at[pl.ds(off, CHUNK)],
                                semo[b])

    def wait_gather(b):
        pltpu.make_async_copy(table_hbm.at[idx_v.at[pl.ds(0, CHUNK)]],
                              rows_v.at[b], semg[b]).wait()

    def wait_scatter(b):
        pltpu.make_async_copy(rows_v.at[b], out_hbm.at[pl.ds(0, CHUNK)],
                              semo[b]).wait()

    for c in range(LOOKAHEAD):
        start_gather(c, c % NBUF)

    def round_body(r, _):
        c0 = r * NBUF
        for b in range(NBUF):
            c = c0 + b
            wait_gather(b)

            @plsc.parallel_loop(0, CHUNK // L)
            def add_body(g):
                lv = lab_v[pl.ds(c * CHUNK + g * L, L)]
                for j in range(L):
                    rr = g * L + j
                    m = lv[j] > 0
                    for jj in range(C // L):
                        s = pl.ds(jj * L, L)
                        rows_v[b, rr, s] = (rows_v[b, rr, s]
                                            + jnp.where(m, e1[jj], e0[jj]))
            start_scatter(c, b)
            nxt = c + LOOKAHEAD
            nb = (b + LOOKAHEAD) % NBUF

            @pl.when(nxt < NCHUNK)
            def _():
                @pl.when(nxt >= NBUF)
                def _():
                    wait_scatter(nb)
                start_gather(nxt, nb)
        return _

    lax.fori_loop(0, NROUND, round_body, None)
    for b in range(NBUF):
        wait_scatter(b)


@jax.jit
def _encode(pts_flat, lab_flat, pemb, table):
    mesh = plsc.VectorSubcoreMesh(core_axis_name="c", subcore_axis_name="s")
    return pl.kernel(
        _body,
        out_type=jax.ShapeDtypeStruct((BN, C), jnp.float32),
        mesh=mesh,
        scratch_types=[
            pltpu.VMEM((BPW,), jnp.int32),
            pltpu.VMEM((BPW,), jnp.int32),
            pltpu.VMEM((BPW,), jnp.int32),
            pltpu.VMEM((BPW,), jnp.int32),
            pltpu.VMEM((BPW,), jnp.int32),
            pltpu.VMEM((2, C), jnp.float32),
            pltpu.VMEM((NBUF, CHUNK, C), jnp.float32),
            [pltpu.SemaphoreType.DMA] * 5,
            [pltpu.SemaphoreType.DMA] * NBUF,
            [pltpu.SemaphoreType.DMA] * NBUF,
        ],
    )(pts_flat, lab_flat, pemb, table)


def kernel(points, labels, point_embeddings, pos_embed):
    B, N = points.shape[0], points.shape[1]
    table = pos_embed.reshape(C, V).T
    pts_flat = points.transpose(2, 0, 1).reshape(3 * B * N).astype(jnp.int32)
    lab_flat = labels.reshape(B * N).astype(jnp.int32)
    out = _encode(pts_flat, lab_flat, point_embeddings, table)
    return out.reshape(B, N, C)


# speedup vs baseline: 1.1934x; 1.1934x over previous
import functools

import jax
import jax.numpy as jnp
from jax import lax
from jax.experimental import pallas as pl
from jax.experimental.pallas import tpu as pltpu
from jax.experimental.pallas import tpu_sc as plsc

C = 256
D = H = W = 64
V = D * H * W
BN = 64 * 512
L = 16
NC, NS = 2, 16
NW = NC * NS
BPW = BN // NW
CHUNK = 64
NCHUNK = BPW // CHUNK
NBUF = 4
NROUND = NCHUNK // NBUF
LOOKAHEAD = 3


def _body(pts_hbm, lab_hbm, pemb_hbm, table_hbm, out_hbm,
          zv, yv, xv, idx_v, lab_v, pemb_v, rows_v, semc, semg, semo):
    wid = lax.axis_index("s") * NC + lax.axis_index("c")
    base = wid * BPW

    cz = pltpu.async_copy(pts_hbm.at[pl.ds(base, BPW)], zv, semc[0])
    cy = pltpu.async_copy(pts_hbm.at[pl.ds(BN + base, BPW)], yv, semc[1])
    cx = pltpu.async_copy(pts_hbm.at[pl.ds(2 * BN + base, BPW)], xv, semc[2])
    cl = pltpu.async_copy(lab_hbm.at[pl.ds(base, BPW)], lab_v, semc[3])
    cp = pltpu.async_copy(pemb_hbm, pemb_v, semc[4])
    cz.wait(); cy.wait(); cx.wait()

    def flat_body(i, _):
        s = pl.ds(i * L, L)
        z = jnp.clip(zv[s], 0, D - 1)
        y = jnp.clip(yv[s], 0, H - 1)
        x = jnp.clip(xv[s], 0, W - 1)
        idx_v[s] = (z * H + y) * W + x
        return _

    def start_gather(c, b):
        off = pl.multiple_of(c * CHUNK, CHUNK)
        idx_slice = idx_v.at[pl.ds(off, CHUNK)]
        return pltpu.async_copy(table_hbm.at[idx_slice], rows_v.at[b], semg[b])

    def start_scatter(c, b):
        off = pl.multiple_of(base + c * CHUNK, CHUNK)
        return pltpu.async_copy(rows_v.at[b], out_hbm.at[pl.ds(off, CHUNK)],
                                semo[b])

    def wait_gather(b):
        pltpu.make_async_copy(table_hbm.at[idx_v.at[pl.ds(0, CHUNK)]],
                              rows_v.at[b], semg[b]).wait()

    def wait_scatter(b):
        pltpu.make_async_copy(rows_v.at[b], out_hbm.at[pl.ds(0, CHUNK)],
                              semo[b]).wait()

    lax.fori_loop(0, LOOKAHEAD * (CHUNK // L), flat_body, None)
    for c in range(LOOKAHEAD):
        start_gather(c, c % NBUF)
    lax.fori_loop(LOOKAHEAD * (CHUNK // L), BPW // L, flat_body, None)
    cl.wait(); cp.wait()

    e0 = [pemb_v[0, pl.ds(j * L, L)] for j in range(C // L)]
    e1 = [pemb_v[1, pl.ds(j * L, L)] for j in range(C // L)]

    def round_body(r, _):
        c0 = r * NBUF
        for b in range(NBUF):
            c = c0 + b
            wait_gather(b)

            @plsc.parallel_loop(0, CHUNK // L)
            def add_body(g):
                lv = lab_v[pl.ds(c * CHUNK + g * L, L)]
                for j in range(L):
                    rr = g * L + j
                    m = lv[j] > 0
                    for jj in range(C // L):
                        s = pl.ds(jj * L, L)
                        rows_v[b, rr, s] = (rows_v[b, rr, s]
                                            + jnp.where(m, e1[jj], e0[jj]))
            start_scatter(c, b)
            nxt = c + LOOKAHEAD
            nb = (b + LOOKAHEAD) % NBUF

            @pl.when(nxt < NCHUNK)
            def _():
                @pl.when(nxt >= NBUF)
                def _():
                    wait_scatter(nb)
                start_gather(nxt, nb)
        return _

    lax.fori_loop(0, NROUND, round_body, None)
    for b in range(NBUF):
        wait_scatter(b)


@jax.jit
def _encode(pts_flat, lab_flat, pemb, table):
    mesh = plsc.VectorSubcoreMesh(core_axis_name="c", subcore_axis_name="s")
    return pl.kernel(
        _body,
        out_type=jax.ShapeDtypeStruct((BN, C), jnp.float32),
        mesh=mesh,
        scratch_types=[
            pltpu.VMEM((BPW,), jnp.int32),
            pltpu.VMEM((BPW,), jnp.int32),
            pltpu.VMEM((BPW,), jnp.int32),
            pltpu.VMEM((BPW,), jnp.int32),
            pltpu.VMEM((BPW,), jnp.int32),
            pltpu.VMEM((2, C), jnp.float32),
            pltpu.VMEM((NBUF, CHUNK, C), jnp.float32),
            [pltpu.SemaphoreType.DMA] * 5,
            [pltpu.SemaphoreType.DMA] * NBUF,
            [pltpu.SemaphoreType.DMA] * NBUF,
        ],
    )(pts_flat, lab_flat, pemb, table)


def kernel(points, labels, point_embeddings, pos_embed):
    B, N = points.shape[0], points.shape[1]
    table = pos_embed.reshape(C, V).T
    pts_flat = points.transpose(2, 0, 1).reshape(3 * B * N).astype(jnp.int32)
    lab_flat = labels.reshape(B * N).astype(jnp.int32)
    out = _encode(pts_flat, lab_flat, point_embeddings, table)
    return out.reshape(B, N, C)


# speedup vs baseline: 1.1964x; 1.0025x over previous
import functools

import jax
import jax.numpy as jnp
from jax import lax
from jax.experimental import pallas as pl
from jax.experimental.pallas import tpu as pltpu
from jax.experimental.pallas import tpu_sc as plsc

C = 256
D = H = W = 64
V = D * H * W
BN = 64 * 512
L = 16
NC, NS = 2, 16
NW = NC * NS
BPW = BN // NW
CHUNK = 64
NCHUNK = BPW // CHUNK
NBUF = 4
NROUND = NCHUNK // NBUF
LOOKAHEAD = 3


def _body(pts_hbm, lab_hbm, pemb_hbm, table_hbm, out_hbm,
          zv, yv, xv, idx_v, lab_v, pemb_v, rows_v, semc, semg, semo):
    wid = lax.axis_index("s") * NC + lax.axis_index("c")
    base = wid * BPW

    cz = pltpu.async_copy(pts_hbm.at[pl.ds(base, BPW)], zv, semc[0])
    cy = pltpu.async_copy(pts_hbm.at[pl.ds(BN + base, BPW)], yv, semc[1])
    cx = pltpu.async_copy(pts_hbm.at[pl.ds(2 * BN + base, BPW)], xv, semc[2])
    cl = pltpu.async_copy(lab_hbm.at[pl.ds(base, BPW)], lab_v, semc[3])
    cp = pltpu.async_copy(pemb_hbm, pemb_v, semc[4])
    cz.wait(); cy.wait(); cx.wait()

    def flat_body(i, _):
        s = pl.ds(i * L, L)
        z = jnp.clip(zv[s], 0, D - 1)
        y = jnp.clip(yv[s], 0, H - 1)
        x = jnp.clip(xv[s], 0, W - 1)
        idx_v[s] = (z * H + y) * W + x
        return _

    def start_gather(c, b):
        off = pl.multiple_of(c * CHUNK, CHUNK)
        idx_slice = idx_v.at[pl.ds(off, CHUNK)]
        return pltpu.async_copy(table_hbm.at[idx_slice], rows_v.at[b], semg[b])

    def start_scatter(c, b):
        off = pl.multiple_of(base + c * CHUNK, CHUNK)
        return pltpu.async_copy(rows_v.at[b], out_hbm.at[pl.ds(off, CHUNK)],
                                semo[b])

    def wait_gather(b):
        pltpu.make_async_copy(table_hbm.at[idx_v.at[pl.ds(0, CHUNK)]],
                              rows_v.at[b], semg[b]).wait()

    def wait_scatter(b):
        pltpu.make_async_copy(rows_v.at[b], out_hbm.at[pl.ds(0, CHUNK)],
                              semo[b]).wait()

    lax.fori_loop(0, LOOKAHEAD * (CHUNK // L), flat_body, None)
    for c in range(LOOKAHEAD):
        start_gather(c, c % NBUF)
    lax.fori_loop(LOOKAHEAD * (CHUNK // L), BPW // L, flat_body, None)
    cl.wait(); cp.wait()

    e0 = [pemb_v[0, pl.ds(j * L, L)] for j in range(C // L)]
    e1 = [pemb_v[1, pl.ds(j * L, L)] for j in range(C // L)]

    def round_body(r, _):
        c0 = r * NBUF
        for b in range(NBUF):
            c = c0 + b
            wait_gather(b)

            @plsc.parallel_loop(0, CHUNK // L)
            def add_body(g):
                lv = lab_v[pl.ds(c * CHUNK + g * L, L)]
                for j in range(L):
                    rr = g * L + j
                    m = lv[j] > 0
                    for jj in range(C // L):
                        s = pl.ds(jj * L, L)
                        rows_v[b, rr, s] = (rows_v[b, rr, s]
                                            + jnp.where(m, e1[jj], e0[jj]))
            start_scatter(c, b)
            nxt = c + LOOKAHEAD
            nb = (b + LOOKAHEAD) % NBUF

            @pl.when(nxt < NCHUNK)
            def _():
                @pl.when(nxt >= NBUF)
                def _():
                    wait_scatter(nb)
                start_gather(nxt, nb)
        return _

    lax.fori_loop(0, NROUND, round_body, None)
    for b in range(NBUF):
        wait_scatter(b)


@jax.jit
def _encode(pts_flat, lab_flat, pemb, table):
    mesh = plsc.VectorSubcoreMesh(core_axis_name="c", subcore_axis_name="s")
    return pl.kernel(
        _body,
        out_type=jax.ShapeDtypeStruct((BN, C), jnp.float32),
        mesh=mesh,
        compiler_params=pltpu.CompilerParams(skip_device_barrier=True),
        scratch_types=[
            pltpu.VMEM((BPW,), jnp.int32),
            pltpu.VMEM((BPW,), jnp.int32),
            pltpu.VMEM((BPW,), jnp.int32),
            pltpu.VMEM((BPW,), jnp.int32),
            pltpu.VMEM((BPW,), jnp.int32),
            pltpu.VMEM((2, C), jnp.float32),
            pltpu.VMEM((NBUF, CHUNK, C), jnp.float32),
            [pltpu.SemaphoreType.DMA] * 5,
            [pltpu.SemaphoreType.DMA] * NBUF,
            [pltpu.SemaphoreType.DMA] * NBUF,
        ],
    )(pts_flat, lab_flat, pemb, table)


def kernel(points, labels, point_embeddings, pos_embed):
    B, N = points.shape[0], points.shape[1]
    table = pos_embed.reshape(C, V).T
    pts_flat = points.transpose(2, 0, 1).reshape(3 * B * N).astype(jnp.int32)
    lab_flat = labels.reshape(B * N).astype(jnp.int32)
    out = _encode(pts_flat, lab_flat, point_embeddings, table)
    return out.reshape(B, N, C)


# speedup vs baseline: 1.2092x; 1.0107x over previous
import functools

import jax
import jax.numpy as jnp
from jax import lax
from jax.experimental import pallas as pl
from jax.experimental.pallas import tpu as pltpu
from jax.experimental.pallas import tpu_sc as plsc

C = 256
D = H = W = 64
V = D * H * W
BN = 64 * 512
L = 16
NC, NS = 2, 16
NW = NC * NS
BPW = BN // NW
CHUNK = 64
NCHUNK = BPW // CHUNK
NBUF = 4
NROUND = NCHUNK // NBUF
LOOKAHEAD = 3


def _body(fidx_hbm, lab_hbm, pemb_hbm, table_hbm, out_hbm,
          idx_v, lab_v, pemb_v, rows_v, semc, semg, semo):
    wid = lax.axis_index("s") * NC + lax.axis_index("c")
    base = wid * BPW

    ci = pltpu.async_copy(fidx_hbm.at[pl.ds(base, BPW)], idx_v, semc[0])
    cl = pltpu.async_copy(lab_hbm.at[pl.ds(base, BPW)], lab_v, semc[1])
    cp = pltpu.async_copy(pemb_hbm, pemb_v, semc[2])
    ci.wait()

    def start_gather(c, b):
        off = pl.multiple_of(c * CHUNK, CHUNK)
        idx_slice = idx_v.at[pl.ds(off, CHUNK)]
        return pltpu.async_copy(table_hbm.at[idx_slice], rows_v.at[b], semg[b])

    def start_scatter(c, b):
        off = pl.multiple_of(base + c * CHUNK, CHUNK)
        return pltpu.async_copy(rows_v.at[b], out_hbm.at[pl.ds(off, CHUNK)],
                                semo[b])

    def wait_gather(b):
        pltpu.make_async_copy(table_hbm.at[idx_v.at[pl.ds(0, CHUNK)]],
                              rows_v.at[b], semg[b]).wait()

    def wait_scatter(b):
        pltpu.make_async_copy(rows_v.at[b], out_hbm.at[pl.ds(0, CHUNK)],
                              semo[b]).wait()

    for c in range(LOOKAHEAD):
        start_gather(c, c % NBUF)
    cl.wait(); cp.wait()

    e0 = [pemb_v[0, pl.ds(j * L, L)] for j in range(C // L)]
    e1 = [pemb_v[1, pl.ds(j * L, L)] for j in range(C // L)]

    def round_body(r, _):
        c0 = r * NBUF
        for b in range(NBUF):
            c = c0 + b
            wait_gather(b)

            @plsc.parallel_loop(0, CHUNK // L)
            def add_body(g):
                lv = lab_v[pl.ds(c * CHUNK + g * L, L)]
                for j in range(L):
                    rr = g * L + j
                    m = lv[j] > 0
                    for jj in range(C // L):
                        s = pl.ds(jj * L, L)
                        rows_v[b, rr, s] = (rows_v[b, rr, s]
                                            + jnp.where(m, e1[jj], e0[jj]))
            start_scatter(c, b)
            nxt = c + LOOKAHEAD
            nb = (b + LOOKAHEAD) % NBUF

            @pl.when(nxt < NCHUNK)
            def _():
                @pl.when(nxt >= NBUF)
                def _():
                    wait_scatter(nb)
                start_gather(nxt, nb)
        return _

    lax.fori_loop(0, NROUND, round_body, None)
    for b in range(NBUF):
        wait_scatter(b)


@jax.jit
def _encode(fidx, lab_flat, pemb, table):
    mesh = plsc.VectorSubcoreMesh(core_axis_name="c", subcore_axis_name="s")
    return pl.kernel(
        _body,
        out_type=jax.ShapeDtypeStruct((BN, C), jnp.float32),
        mesh=mesh,
        compiler_params=pltpu.CompilerParams(skip_device_barrier=True),
        scratch_types=[
            pltpu.VMEM((BPW,), jnp.int32),
            pltpu.VMEM((BPW,), jnp.int32),
            pltpu.VMEM((2, C), jnp.float32),
            pltpu.VMEM((NBUF, CHUNK, C), jnp.float32),
            [pltpu.SemaphoreType.DMA] * 3,
            [pltpu.SemaphoreType.DMA] * NBUF,
            [pltpu.SemaphoreType.DMA] * NBUF,
        ],
    )(fidx, lab_flat, pemb, table)


def kernel(points, labels, point_embeddings, pos_embed):
    B, N = points.shape[0], points.shape[1]
    table = pos_embed.reshape(C, V).T
    pts = points.astype(jnp.int32)
    z = jnp.clip(pts[..., 0], 0, D - 1)
    y = jnp.clip(pts[..., 1], 0, H - 1)
    x = jnp.clip(pts[..., 2], 0, W - 1)
    fidx = ((z * H + y) * W + x).reshape(B * N)
    lab_flat = labels.reshape(B * N).astype(jnp.int32)
    out = _encode(fidx, lab_flat, point_embeddings, table)
    return out.reshape(B, N, C)
